# baseline (speedup 1.0000x reference)
import functools

import jax
import jax.numpy as jnp
from jax import lax
from jax.experimental import pallas as pl
from jax.experimental.pallas import tpu as pltpu
from jax.experimental.pallas import tpu_sc as plsc

LANE_BLK = 128


@functools.lru_cache(maxsize=None)
def _make_gather(Bt, H, D):
    info = plsc.get_sparse_core_info()
    NC, NS = info.num_cores, info.num_subcores
    NW = NC * NS
    assert Bt % LANE_BLK == 0 and Bt // LANE_BLK == NW and D % 8 == 0
    JG = D // 8
    mesh = plsc.VectorSubcoreMesh(core_axis_name="c", subcore_axis_name="s")

    @functools.partial(
        pl.kernel,
        mesh=mesh,
        out_type=jax.ShapeDtypeStruct((H, JG, NW, 8, LANE_BLK), jnp.float32),
        scratch_types=[
            pltpu.VMEM((2, LANE_BLK), jnp.int32),
            pltpu.VMEM((2, LANE_BLK, D), jnp.float32),
            pltpu.VMEM((2, JG, 8, LANE_BLK), jnp.float32),
            pltpu.SemaphoreType.DMA((2,)),
            pltpu.SemaphoreType.DMA((2,)),
        ],
        compiler_params=pltpu.CompilerParams(
            use_tc_tiling_on_sc=False, needs_layout_passes=False
        ),
    )
    def gather_kernel(idxT_hbm, table_hbm, out_hbm, idx_v, rows_v, tiles_v, gsem, wsem):
        wid = lax.axis_index("s") * NC + lax.axis_index("c")
        iota = lax.iota(jnp.int32, 16)

        def idx_load(h, p):
            pltpu.sync_copy(
                idxT_hbm.at[pl.ds(h * Bt + wid * LANE_BLK, LANE_BLK)], idx_v.at[p]
            )

        def gather_start(p):
            pltpu.async_copy(table_hbm.at[idx_v.at[p]], rows_v.at[p], gsem.at[p])

        def gather_wait(p):
            pltpu.make_async_copy(
                table_hbm.at[idx_v.at[p]], rows_v.at[p], gsem.at[p]
            ).wait()

        def write_start(h, p):
            pltpu.async_copy(
                tiles_v.at[p], out_hbm.at[h, slice(None), wid], wsem.at[p]
            )

        def write_wait(p):
            pltpu.make_async_copy(
                tiles_v.at[p], out_hbm.at[0, slice(None), wid], wsem.at[p]
            ).wait()

        def transpose(p):
            for c in range(D):
                for k in range(LANE_BLK // 16):
                    v = plsc.load_gather(
                        rows_v.at[p],
                        [iota + (16 * k), jnp.full((16,), c, jnp.int32)],
                    )
                    tiles_v[p, c // 8, c % 8, pl.ds(k * 16, 16)] = v

        idx_load(0, 0)
        gather_start(0)
        idx_load(1, 1)
        gather_start(1)

        def step(h, carry):
            for p in (0, 1):
                hh = h + p
                gather_wait(p)

                @pl.when(hh >= 2)
                def _():
                    write_wait(p)

                transpose(p)
                write_start(hh, p)

                @pl.when(hh + 2 < H)
                def _():
                    idx_load(hh + 2, p)
                    gather_start(p)

            return carry

        lax.fori_loop(0, H // 2, lambda o, c: step(o * 2, c), 0)
        write_wait(0)
        write_wait(1)

    return gather_kernel


def kernel(input, emb):
    Bt, H = input.shape
    D = emb.shape[1]
    idxT = input.T.astype(jnp.int32).reshape(H * Bt)
    outT = _make_gather(Bt, H, D)(idxT, emb)
    return outT.transpose(2, 4, 0, 1, 3).reshape(Bt, H, D)


# speedup vs baseline: 1.1507x; 1.1507x over previous
import functools

import jax
import jax.numpy as jnp
from jax import lax
from jax.experimental import pallas as pl
from jax.experimental.pallas import tpu as pltpu
from jax.experimental.pallas import tpu_sc as plsc

LB = 128


@functools.lru_cache(maxsize=None)
def _make_gather(Bt, H, D):
    info = plsc.get_sparse_core_info()
    NC, NS = info.num_cores, info.num_subcores
    NW = NC * NS
    assert Bt % LB == 0 and Bt // LB == NW and D % 8 == 0 and H % 2 == 0
    JG = D // 8
    TILE_F = 8 * LB
    mesh = plsc.VectorSubcoreMesh(core_axis_name="c", subcore_axis_name="s")

    @functools.partial(
        pl.kernel,
        mesh=mesh,
        out_type=jax.ShapeDtypeStruct((H, JG, NW, TILE_F), jnp.float32),
        scratch_types=[
            pltpu.VMEM((H, LB), jnp.int32),
            pltpu.VMEM((2, LB, D), jnp.float32),
            pltpu.VMEM((2, JG * TILE_F), jnp.float32),
            pltpu.SemaphoreType.DMA((2,)),
            pltpu.SemaphoreType.DMA((2,)),
        ],
        compiler_params=pltpu.CompilerParams(
            use_tc_tiling_on_sc=False, needs_layout_passes=False
        ),
    )
    def gather_kernel(idxT_hbm, table_hbm, out_hbm, idx_v, rows_v, tiles_v, gsem, wsem):
        wid = lax.axis_index("s") * NC + lax.axis_index("c")
        iota = lax.iota(jnp.int32, 16)
        pats = [
            ((16 * k + iota) // 8) * TILE_F + ((16 * k + iota) % 8) * LB
            for k in range(D // 16)
        ]

        pltpu.sync_copy(idxT_hbm.at[:, pl.ds(wid * LB, LB)], idx_v)

        def gather_start(h, p):
            pltpu.async_copy(table_hbm.at[idx_v.at[h]], rows_v.at[p], gsem.at[p])

        def gather_wait(p):
            pltpu.make_async_copy(
                table_hbm.at[idx_v.at[0]], rows_v.at[p], gsem.at[p]
            ).wait()

        def write_start(h, p):
            for jg in range(JG):
                pltpu.async_copy(
                    tiles_v.at[p, pl.ds(jg * TILE_F, TILE_F)],
                    out_hbm.at[h, jg, wid],
                    wsem.at[p],
                )

        def write_wait(p):
            for jg in range(JG):
                pltpu.make_async_copy(
                    tiles_v.at[p, pl.ds(jg * TILE_F, TILE_F)],
                    out_hbm.at[0, jg, wid],
                    wsem.at[p],
                ).wait()

        def transpose(p):
            for bl in range(LB):
                for k in range(D // 16):
                    v = rows_v[p, bl, pl.ds(16 * k, 16)]
                    plsc.store_scatter(tiles_v.at[p], [pats[k] + bl], v)

        gather_start(0, 0)
        gather_start(1, 1)

        def step(h, carry):
            for p in (0, 1):
                hh = h + p
                gather_wait(p)

                @pl.when(hh >= 2)
                def _():
                    write_wait(p)

                transpose(p)
                write_start(hh, p)

                @pl.when(hh + 2 < H)
                def _():
                    gather_start(hh + 2, p)

            return carry

        lax.fori_loop(0, H // 2, lambda o, c: step(o * 2, c), 0)
        write_wait(0)
        write_wait(1)

    return gather_kernel


def kernel(input, emb):
    Bt, H = input.shape
    D = emb.shape[1]
    idxT = input.T.astype(jnp.int32)
    outT = _make_gather(Bt, H, D)(idxT, emb)
    return (
        outT.reshape(H, D // 8, Bt // LB, 8, LB)
        .transpose(2, 4, 0, 1, 3)
        .reshape(Bt, H, D)
    )


# speedup vs baseline: 1.2369x; 1.0749x over previous
import functools

import jax
import jax.numpy as jnp
from jax import lax
from jax.experimental import pallas as pl
from jax.experimental.pallas import tpu as pltpu
from jax.experimental.pallas import tpu_sc as plsc

LB = 128
DET_COLS = 3200


@functools.lru_cache(maxsize=None)
def _make_detile(V, D):
    n_steps = (V + DET_COLS - 1) // DET_COLS

    def body(x_ref, o_ref):
        t = x_ref[...].T.reshape(DET_COLS // 2, 2, D)
        o_ref[:, 0:D] = t[:, 0, :]
        o_ref[:, D : 2 * D] = t[:, 1, :]

    return pl.pallas_call(
        body,
        grid=(n_steps,),
        in_specs=[pl.BlockSpec((D, DET_COLS), lambda i: (0, i))],
        out_specs=pl.BlockSpec((DET_COLS // 2, 2 * D), lambda i: (i, 0)),
        out_shape=jax.ShapeDtypeStruct((V // 2, 2 * D), jnp.float32),
        compiler_params=pltpu.CompilerParams(
            dimension_semantics=("arbitrary",)
        ),
    )


@functools.lru_cache(maxsize=None)
def _make_gather(Bt, H, D):
    info = plsc.get_sparse_core_info()
    NC, NS = info.num_cores, info.num_subcores
    NW = NC * NS
    assert Bt % LB == 0 and Bt // LB == NW and D % 8 == 0 and H % 2 == 0
    JG = D // 8
    TILE_F = 8 * LB
    D2 = 2 * D
    mesh = plsc.VectorSubcoreMesh(core_axis_name="c", subcore_axis_name="s")

    @functools.partial(
        pl.kernel,
        mesh=mesh,
        out_type=jax.ShapeDtypeStruct((H, JG, NW, TILE_F), jnp.float32),
        scratch_types=[
            pltpu.VMEM((H, LB), jnp.int32),
            pltpu.VMEM((H, LB), jnp.int32),
            pltpu.VMEM((2, LB), jnp.int32),
            pltpu.VMEM((2, LB, D2), jnp.float32),
            pltpu.VMEM((2, JG * TILE_F), jnp.float32),
            pltpu.SemaphoreType.DMA((2,)),
            pltpu.SemaphoreType.DMA((2,)),
        ],
        compiler_params=pltpu.CompilerParams(
            use_tc_tiling_on_sc=False, needs_layout_passes=False
        ),
    )
    def gather_kernel(
        idxT_hbm, table_hbm, out_hbm, idx_v, pair_v, cur_v, rows_v, tiles_v, gsem, wsem
    ):
        wid = lax.axis_index("s") * NC + lax.axis_index("c")
        iota = lax.iota(jnp.int32, 16)

        pltpu.sync_copy(idxT_hbm.at[:, pl.ds(wid * LB, LB)], idx_v)
        for h in range(H):
            for k in range(LB // 16):
                pair_v[h, pl.ds(16 * k, 16)] = (
                    idx_v[h, pl.ds(16 * k, 16)] >> 1
                )

        def gather_start(h, p):
            pltpu.async_copy(table_hbm.at[pair_v.at[h]], rows_v.at[p], gsem.at[p])

        def gather_wait(p):
            pltpu.make_async_copy(
                table_hbm.at[pair_v.at[0]], rows_v.at[p], gsem.at[p]
            ).wait()

        def write_start(h, p):
            for jg in range(JG):
                pltpu.async_copy(
                    tiles_v.at[p, pl.ds(jg * TILE_F, TILE_F)],
                    out_hbm.at[h, jg, wid],
                    wsem.at[p],
                )

        def write_wait(p):
            for jg in range(JG):
                pltpu.make_async_copy(
                    tiles_v.at[p, pl.ds(jg * TILE_F, TILE_F)],
                    out_hbm.at[0, jg, wid],
                    wsem.at[p],
                ).wait()

        def transpose(p):
            base = []
            for k in range(LB // 16):
                par = cur_v[p, pl.ds(16 * k, 16)] & 1
                base.append(par * D)
            blv = [iota + 16 * k for k in range(LB // 16)]

            def col(c, carry):
                off = (c // 8) * TILE_F + (c % 8) * LB
                for k in range(LB // 16):
                    v = plsc.load_gather(rows_v.at[p], [blv[k], base[k] + c])
                    tiles_v[p, pl.ds(off + 16 * k, 16)] = v
                return carry

            lax.fori_loop(0, D, col, 0)

        gather_start(0, 0)
        gather_start(1, 1)

        def step(h, carry):
            for p in (0, 1):
                hh = h + p
                gather_wait(p)
                pltpu.sync_copy(
                    idxT_hbm.at[hh, pl.ds(wid * LB, LB)], cur_v.at[p]
                )

                @pl.when(hh >= 2)
                def _():
                    write_wait(p)

                transpose(p)
                write_start(hh, p)

                @pl.when(hh + 2 < H)
                def _():
                    gather_start(hh + 2, p)

            return carry

        lax.fori_loop(0, H // 2, lambda o, c: step(o * 2, c), 0)
        write_wait(0)
        write_wait(1)

    return gather_kernel


def kernel(input, emb):
    Bt, H = input.shape
    V, D = emb.shape
    table2 = _make_detile(V, D)(emb.T)
    idxT = input.T.astype(jnp.int32)
    outT = _make_gather(Bt, H, D)(idxT, table2)
    return (
        outT.reshape(H, D // 8, Bt // LB, 8, LB)
        .transpose(2, 4, 0, 1, 3)
        .reshape(Bt, H, D)
    )


# speedup vs baseline: 1.3161x; 1.0641x over previous
import functools

import jax
import jax.numpy as jnp
from jax import lax
from jax.experimental import pallas as pl
from jax.experimental.pallas import tpu as pltpu
from jax.experimental.pallas import tpu_sc as plsc

LB = 128
DET_COLS = 6400


@functools.lru_cache(maxsize=None)
def _make_detile(V, D):
    n_steps = (V + DET_COLS - 1) // DET_COLS

    def body(x_ref, o_ref):
        t = x_ref[...].T.reshape(DET_COLS // 2, 2, D)
        o_ref[:, 0:D] = t[:, 0, :]
        o_ref[:, D : 2 * D] = t[:, 1, :]

    return pl.pallas_call(
        body,
        grid=(n_steps,),
        in_specs=[pl.BlockSpec((D, DET_COLS), lambda i: (0, i))],
        out_specs=pl.BlockSpec((DET_COLS // 2, 2 * D), lambda i: (i, 0)),
        out_shape=jax.ShapeDtypeStruct((V // 2, 2 * D), jnp.float32),
        compiler_params=pltpu.CompilerParams(
            dimension_semantics=("arbitrary",)
        ),
    )


@functools.lru_cache(maxsize=None)
def _make_gather(Bt, H, D):
    info = plsc.get_sparse_core_info()
    NC, NS = info.num_cores, info.num_subcores
    NW = NC * NS
    assert Bt % LB == 0 and Bt // LB == NW and D % 8 == 0 and H % 2 == 0
    JG = D // 8
    TILE_F = 8 * LB
    D2 = 2 * D
    mesh = plsc.VectorSubcoreMesh(core_axis_name="c", subcore_axis_name="s")

    @functools.partial(
        pl.kernel,
        mesh=mesh,
        out_type=jax.ShapeDtypeStruct((H, JG, NW, TILE_F), jnp.float32),
        scratch_types=[
            pltpu.VMEM((H, LB), jnp.int32),
            pltpu.VMEM((H, LB), jnp.int32),
            pltpu.VMEM((2, LB), jnp.int32),
            pltpu.VMEM((2, LB, D2), jnp.float32),
            pltpu.VMEM((2, JG * TILE_F), jnp.float32),
            pltpu.SemaphoreType.DMA((2,)),
            pltpu.SemaphoreType.DMA((2,)),
        ],
        compiler_params=pltpu.CompilerParams(
            use_tc_tiling_on_sc=False, needs_layout_passes=False
        ),
    )
    def gather_kernel(
        idxT_hbm, table_hbm, out_hbm, idx_v, pair_v, cur_v, rows_v, tiles_v, gsem, wsem
    ):
        wid = lax.axis_index("s") * NC + lax.axis_index("c")
        iota = lax.iota(jnp.int32, 16)

        pltpu.sync_copy(idxT_hbm.at[:, pl.ds(wid * LB, LB)], idx_v)

        def mkpair(h, carry):
            for k in range(LB // 16):
                pair_v[h, pl.ds(16 * k, 16)] = idx_v[h, pl.ds(16 * k, 16)] >> 1
            return carry

        lax.fori_loop(0, H, mkpair, 0)

        def gather_start(h, p):
            pltpu.async_copy(table_hbm.at[pair_v.at[h]], rows_v.at[p], gsem.at[p])

        def gather_wait(p):
            pltpu.make_async_copy(
                table_hbm.at[pair_v.at[0]], rows_v.at[p], gsem.at[p]
            ).wait()

        def write_start(h, p):
            for jg in range(JG):
                pltpu.async_copy(
                    tiles_v.at[p, pl.ds(jg * TILE_F, TILE_F)],
                    out_hbm.at[h, jg, wid],
                    wsem.at[p],
                )

        def write_wait(p):
            for jg in range(JG):
                pltpu.make_async_copy(
                    tiles_v.at[p, pl.ds(jg * TILE_F, TILE_F)],
                    out_hbm.at[0, jg, wid],
                    wsem.at[p],
                ).wait()

        def transpose(p):
            base = []
            for k in range(LB // 16):
                par = cur_v[p, pl.ds(16 * k, 16)] & 1
                base.append(par * D)
            blv = [iota + 16 * k for k in range(LB // 16)]

            def col8(c0, carry):
                for dc in range(8):
                    c = c0 * 8 + dc
                    off = (c // 8) * TILE_F + (c % 8) * LB
                    for k in range(LB // 16):
                        v = plsc.load_gather(
                            rows_v.at[p], [blv[k], base[k] + c]
                        )
                        tiles_v[p, pl.ds(off + 16 * k, 16)] = v
                return carry

            lax.fori_loop(0, D // 8, col8, 0)

        gather_start(0, 0)
        gather_start(1, 1)

        def step(h, carry):
            for p in (0, 1):
                hh = h + p
                gather_wait(p)
                pltpu.sync_copy(
                    idxT_hbm.at[hh, pl.ds(wid * LB, LB)], cur_v.at[p]
                )

                @pl.when(hh >= 2)
                def _():
                    write_wait(p)

                transpose(p)
                write_start(hh, p)

                @pl.when(hh + 2 < H)
                def _():
                    gather_start(hh + 2, p)

            return carry

        lax.fori_loop(0, H // 2, lambda o, c: step(o * 2, c), 0)
        write_wait(0)
        write_wait(1)

    return gather_kernel


def kernel(input, emb):
    Bt, H = input.shape
    V, D = emb.shape
    table2 = _make_detile(V, D)(emb.T)
    idxT = input.T.astype(jnp.int32)
    outT = _make_gather(Bt, H, D)(idxT, table2)
    return (
        outT.reshape(H, D // 8, Bt // LB, 8, LB)
        .transpose(2, 4, 0, 1, 3)
        .reshape(Bt, H, D)
    )
